# baseline (speedup 1.0000x reference)
import functools

import jax
import jax.numpy as jnp
from jax import lax
from jax.experimental import pallas as pl
from jax.experimental.pallas import tpu as pltpu
from jax.experimental.pallas import tpu_sc as plsc

BATCH = 4096
NUM_FIELDS = 26
EMBED_DIM = 16
VOCAB = 1000000

NUM_CORES = 2
NUM_SUBCORES = 16
NW = NUM_CORES * NUM_SUBCORES
CHUNK = 128
NBLK = BATCH // CHUNK

RUN = 1024
NFULL = VOCAB // RUN
TAIL_ROWS = 512
LAST64 = VOCAB - NFULL * RUN - TAIL_ROWS
RPW = NFULL // NW
NFREM = NFULL % NW

_mesh = plsc.VectorSubcoreMesh(core_axis_name="c", subcore_axis_name="s")


@functools.partial(
    pl.kernel,
    mesh=_mesh,
    out_type=jax.ShapeDtypeStruct((VOCAB * EMBED_DIM,), jnp.float32),
    scratch_types=[
        pltpu.VMEM((EMBED_DIM, RUN), jnp.float32),
        pltpu.VMEM((EMBED_DIM, RUN), jnp.float32),
        pltpu.VMEM((RUN * EMBED_DIM,), jnp.float32),
        pltpu.VMEM((RUN * EMBED_DIM,), jnp.float32),
        pltpu.SemaphoreType.DMA,
        pltpu.SemaphoreType.DMA,
        pltpu.SemaphoreType.DMA,
    ],
    compiler_params=pltpu.CompilerParams(use_tc_tiling_on_sc=True,
                                         needs_layout_passes=False),
)
def _transpose_kernel(tab_hbm, tail_hbm, out_hbm, in_a, in_b, ob_a, ob_b,
                      sem_a, sem_b, sem_o):
    w = lax.axis_index("s") * NUM_CORES + lax.axis_index("c")
    nruns = jnp.where(w < NFREM, RPW + 1, RPW)

    iota = lax.iota(jnp.int32, 16)

    def fire_load(k, ibuf, sem):
        r0 = (w + k * NW) * RUN
        return [
            pltpu.async_copy(tab_hbm.at[pl.ds(0, 8), pl.ds(r0, RUN)],
                             ibuf.at[pl.ds(0, 8)], sem),
            pltpu.async_copy(tab_hbm.at[pl.ds(8, 8), pl.ds(r0, RUN)],
                             ibuf.at[pl.ds(8, 8)], sem),
        ]

    def transpose_run(ibuf, obuf, nrows):
        def body(j, cvec):
            g0 = plsc.load_gather(ibuf, [iota, cvec])
            g1 = plsc.load_gather(ibuf, [iota, cvec + 1])
            obuf[pl.ds(j * 2 * EMBED_DIM, 16)] = g0
            obuf[pl.ds((j * 2 + 1) * EMBED_DIM, 16)] = g1
            return cvec + 2

        lax.fori_loop(0, nrows // 2, body, jnp.zeros((16,), jnp.int32),
                      unroll=8)

    def write_run(k, obuf, nrows):
        r0 = (w + k * NW) * RUN
        cp = pltpu.async_copy(
            obuf.at[pl.ds(0, nrows * EMBED_DIM)],
            out_hbm.at[pl.ds(r0 * EMBED_DIM, nrows * EMBED_DIM)],
            sem_o)
        cp.wait()

    def half(k, ibuf, obuf, sem_cur, ibuf_nxt, sem_nxt):
        @pl.when(k + 1 < nruns)
        def _():
            fire_load(k + 1, ibuf_nxt, sem_nxt)

        @pl.when(k < nruns)
        def _():
            pltpu.make_async_copy(
                tab_hbm.at[pl.ds(0, 8), pl.ds(0, RUN)],
                ibuf.at[pl.ds(0, 8)], sem_cur).wait()
            pltpu.make_async_copy(
                tab_hbm.at[pl.ds(8, 8), pl.ds(0, RUN)],
                ibuf.at[pl.ds(8, 8)], sem_cur).wait()
            transpose_run(ibuf, obuf, RUN)
            write_run(k, obuf, RUN)

    fire_load(0, in_a, sem_a)

    def loop(i, _):
        k = i * 2
        half(k, in_a, ob_a, sem_a, in_b, sem_b)
        half(k + 1, in_b, ob_b, sem_b, in_a, sem_a)
        return 0

    lax.fori_loop(0, (RPW + 2) // 2, loop, 0)

    @pl.when(w == NW - 1)
    def _tail():
        r0 = NFULL * RUN
        cps = [
            pltpu.async_copy(
                tab_hbm.at[pl.ds(0, 8), pl.ds(r0, TAIL_ROWS)],
                in_a.at[pl.ds(0, 8), pl.ds(0, TAIL_ROWS)], sem_a),
            pltpu.async_copy(
                tab_hbm.at[pl.ds(8, 8), pl.ds(r0, TAIL_ROWS)],
                in_a.at[pl.ds(8, 8), pl.ds(0, TAIL_ROWS)], sem_a),
        ]
        for cp in cps:
            cp.wait()
        transpose_run(in_a, ob_a, TAIL_ROWS)
        cp = pltpu.async_copy(
            ob_a.at[pl.ds(0, TAIL_ROWS * EMBED_DIM)],
            out_hbm.at[pl.ds(r0 * EMBED_DIM, TAIL_ROWS * EMBED_DIM)],
            sem_o)
        cp.wait()
        pltpu.sync_copy(tail_hbm, ob_b.at[pl.ds(0, LAST64 * EMBED_DIM)])
        pltpu.sync_copy(
            ob_b.at[pl.ds(0, LAST64 * EMBED_DIM)],
            out_hbm.at[pl.ds((VOCAB - LAST64) * EMBED_DIM,
                             LAST64 * EMBED_DIM)])


@functools.partial(
    pl.kernel,
    mesh=_mesh,
    out_type=jax.ShapeDtypeStruct((NUM_FIELDS, 2, NBLK, 8, CHUNK),
                                  jnp.float32),
    scratch_types=[
        pltpu.VMEM((NUM_FIELDS, CHUNK), jnp.int32),
        pltpu.VMEM((NUM_FIELDS, CHUNK, EMBED_DIM), jnp.float32),
        pltpu.VMEM((EMBED_DIM, CHUNK), jnp.float32),
        pltpu.SemaphoreType.DMA,
        pltpu.SemaphoreType.DMA,
    ],
    compiler_params=pltpu.CompilerParams(use_tc_tiling_on_sc=False,
                                         needs_layout_passes=False),
)
def _gather_kernel(idx_hbm, table_hbm, out_hbm,
                   idx_all, rows_all, obuf, sem_i, sem_g):
    w = lax.axis_index("s") * NUM_CORES + lax.axis_index("c")
    col = w * CHUNK

    idx_copies = [
        pltpu.async_copy(idx_hbm.at[f, pl.ds(col, CHUNK)],
                         idx_all.at[f], sem_i)
        for f in range(NUM_FIELDS)
    ]
    for cp in idx_copies:
        cp.wait()

    row_copies = [
        pltpu.async_copy(table_hbm.at[idx_all.at[f]],
                         rows_all.at[f], sem_g)
        for f in range(NUM_FIELDS)
    ]
    for cp in row_copies:
        cp.wait()

    lanes = lax.iota(jnp.int32, 16)

    def field_body(f, _):
        def col_body(c, _):
            cvec = jnp.zeros((16,), jnp.int32) + c
            for k in range(CHUNK // 16):
                v = plsc.load_gather(rows_all.at[f],
                                     [k * 16 + lanes, cvec])
                obuf[c, pl.ds(k * 16, 16)] = v
            return 0
        lax.fori_loop(0, EMBED_DIM, col_body, 0)
        pltpu.sync_copy(obuf.at[pl.ds(0, 8)], out_hbm.at[f, 0, w])
        pltpu.sync_copy(obuf.at[pl.ds(8, 8)], out_hbm.at[f, 1, w])
        return 0

    lax.fori_loop(0, NUM_FIELDS, field_body, 0)


def kernel(sparse_input, table):
    tail_flat = table[VOCAB - LAST64:].reshape(LAST64 * EMBED_DIM)
    table_rm = _transpose_kernel(table.T, tail_flat).reshape(VOCAB,
                                                             EMBED_DIM)
    idx_t = sparse_input.T
    out6 = _gather_kernel(idx_t, table_rm)
    return out6.transpose(2, 4, 0, 1, 3).reshape(BATCH, NUM_FIELDS,
                                                 EMBED_DIM)


# speedup vs baseline: 1.0004x; 1.0004x over previous
import functools

import jax
import jax.numpy as jnp
from jax import lax
from jax.experimental import pallas as pl
from jax.experimental.pallas import tpu as pltpu
from jax.experimental.pallas import tpu_sc as plsc

BATCH = 4096
NUM_FIELDS = 26
EMBED_DIM = 16
VOCAB = 1000000

NUM_CORES = 2
NUM_SUBCORES = 16
NW = NUM_CORES * NUM_SUBCORES
CHUNK = 128
NBLK = BATCH // CHUNK

RUN = 1024
NFULL = VOCAB // RUN
TAIL_ROWS = 512
LAST64 = VOCAB - NFULL * RUN - TAIL_ROWS
RPW = NFULL // NW
NFREM = NFULL % NW

_mesh = plsc.VectorSubcoreMesh(core_axis_name="c", subcore_axis_name="s")


@functools.partial(
    pl.kernel,
    mesh=_mesh,
    out_type=jax.ShapeDtypeStruct((VOCAB * EMBED_DIM,), jnp.float32),
    scratch_types=[
        pltpu.VMEM((EMBED_DIM, RUN + 8), jnp.float32),
        pltpu.VMEM((EMBED_DIM, RUN + 8), jnp.float32),
        pltpu.VMEM((RUN * EMBED_DIM,), jnp.float32),
        pltpu.VMEM((RUN * EMBED_DIM,), jnp.float32),
        pltpu.SemaphoreType.DMA,
        pltpu.SemaphoreType.DMA,
        pltpu.SemaphoreType.DMA,
    ],
    compiler_params=pltpu.CompilerParams(use_tc_tiling_on_sc=True,
                                         needs_layout_passes=False),
)
def _transpose_kernel(tab_hbm, tail_hbm, out_hbm, in_a, in_b, ob_a, ob_b,
                      sem_a, sem_b, sem_o):
    w = lax.axis_index("s") * NUM_CORES + lax.axis_index("c")
    nruns = jnp.where(w < NFREM, RPW + 1, RPW)

    iota = lax.iota(jnp.int32, 16)

    def fire_load(k, ibuf, sem):
        r0 = (w + k * NW) * RUN
        return [
            pltpu.async_copy(tab_hbm.at[pl.ds(0, 8), pl.ds(r0, RUN)],
                             ibuf.at[pl.ds(0, 8), pl.ds(0, RUN)], sem),
            pltpu.async_copy(tab_hbm.at[pl.ds(8, 8), pl.ds(r0, RUN)],
                             ibuf.at[pl.ds(8, 8), pl.ds(0, RUN)], sem),
        ]

    def transpose_run(ibuf, obuf, nrows):
        def body(j, cvec):
            g0 = plsc.load_gather(ibuf, [iota, cvec])
            g1 = plsc.load_gather(ibuf, [iota, cvec + 1])
            obuf[pl.ds(j * 2 * EMBED_DIM, 16)] = g0
            obuf[pl.ds((j * 2 + 1) * EMBED_DIM, 16)] = g1
            return cvec + 2

        lax.fori_loop(0, nrows // 2, body, jnp.zeros((16,), jnp.int32),
                      unroll=8)

    def write_run(k, obuf, nrows):
        r0 = (w + k * NW) * RUN
        cp = pltpu.async_copy(
            obuf.at[pl.ds(0, nrows * EMBED_DIM)],
            out_hbm.at[pl.ds(r0 * EMBED_DIM, nrows * EMBED_DIM)],
            sem_o)
        cp.wait()

    def half(k, ibuf, obuf, sem_cur, ibuf_nxt, sem_nxt):
        @pl.when(k + 1 < nruns)
        def _():
            fire_load(k + 1, ibuf_nxt, sem_nxt)

        @pl.when(k < nruns)
        def _():
            pltpu.make_async_copy(
                tab_hbm.at[pl.ds(0, 8), pl.ds(0, RUN)],
                ibuf.at[pl.ds(0, 8), pl.ds(0, RUN)], sem_cur).wait()
            pltpu.make_async_copy(
                tab_hbm.at[pl.ds(8, 8), pl.ds(0, RUN)],
                ibuf.at[pl.ds(8, 8), pl.ds(0, RUN)], sem_cur).wait()
            transpose_run(ibuf, obuf, RUN)
            write_run(k, obuf, RUN)

    fire_load(0, in_a, sem_a)

    def loop(i, _):
        k = i * 2
        half(k, in_a, ob_a, sem_a, in_b, sem_b)
        half(k + 1, in_b, ob_b, sem_b, in_a, sem_a)
        return 0

    lax.fori_loop(0, (RPW + 2) // 2, loop, 0)

    @pl.when(w == NW - 1)
    def _tail():
        r0 = NFULL * RUN
        cps = [
            pltpu.async_copy(
                tab_hbm.at[pl.ds(0, 8), pl.ds(r0, TAIL_ROWS)],
                in_a.at[pl.ds(0, 8), pl.ds(0, TAIL_ROWS)], sem_a),
            pltpu.async_copy(
                tab_hbm.at[pl.ds(8, 8), pl.ds(r0, TAIL_ROWS)],
                in_a.at[pl.ds(8, 8), pl.ds(0, TAIL_ROWS)], sem_a),
        ]
        for cp in cps:
            cp.wait()
        transpose_run(in_a, ob_a, TAIL_ROWS)
        cp = pltpu.async_copy(
            ob_a.at[pl.ds(0, TAIL_ROWS * EMBED_DIM)],
            out_hbm.at[pl.ds(r0 * EMBED_DIM, TAIL_ROWS * EMBED_DIM)],
            sem_o)
        cp.wait()
        pltpu.sync_copy(tail_hbm, ob_b.at[pl.ds(0, LAST64 * EMBED_DIM)])
        pltpu.sync_copy(
            ob_b.at[pl.ds(0, LAST64 * EMBED_DIM)],
            out_hbm.at[pl.ds((VOCAB - LAST64) * EMBED_DIM,
                             LAST64 * EMBED_DIM)])


@functools.partial(
    pl.kernel,
    mesh=_mesh,
    out_type=jax.ShapeDtypeStruct((NUM_FIELDS, 2, NBLK, 8, CHUNK),
                                  jnp.float32),
    scratch_types=[
        pltpu.VMEM((NUM_FIELDS, CHUNK), jnp.int32),
        pltpu.VMEM((NUM_FIELDS, CHUNK, EMBED_DIM), jnp.float32),
        pltpu.VMEM((EMBED_DIM, CHUNK), jnp.float32),
        pltpu.SemaphoreType.DMA,
        pltpu.SemaphoreType.DMA,
    ],
    compiler_params=pltpu.CompilerParams(use_tc_tiling_on_sc=False,
                                         needs_layout_passes=False),
)
def _gather_kernel(idx_hbm, table_hbm, out_hbm,
                   idx_all, rows_all, obuf, sem_i, sem_g):
    w = lax.axis_index("s") * NUM_CORES + lax.axis_index("c")
    col = w * CHUNK

    idx_copies = [
        pltpu.async_copy(idx_hbm.at[f, pl.ds(col, CHUNK)],
                         idx_all.at[f], sem_i)
        for f in range(NUM_FIELDS)
    ]
    for cp in idx_copies:
        cp.wait()

    row_copies = [
        pltpu.async_copy(table_hbm.at[idx_all.at[f]],
                         rows_all.at[f], sem_g)
        for f in range(NUM_FIELDS)
    ]
    for cp in row_copies:
        cp.wait()

    lanes = lax.iota(jnp.int32, 16)

    def field_body(f, _):
        def col_body(c, _):
            cvec = jnp.zeros((16,), jnp.int32) + c
            for k in range(CHUNK // 16):
                v = plsc.load_gather(rows_all.at[f],
                                     [k * 16 + lanes, cvec])
                obuf[c, pl.ds(k * 16, 16)] = v
            return 0
        lax.fori_loop(0, EMBED_DIM, col_body, 0)
        pltpu.sync_copy(obuf.at[pl.ds(0, 8)], out_hbm.at[f, 0, w])
        pltpu.sync_copy(obuf.at[pl.ds(8, 8)], out_hbm.at[f, 1, w])
        return 0

    lax.fori_loop(0, NUM_FIELDS, field_body, 0)


def kernel(sparse_input, table):
    tail_flat = table[VOCAB - LAST64:].reshape(LAST64 * EMBED_DIM)
    table_rm = _transpose_kernel(table.T, tail_flat).reshape(VOCAB,
                                                             EMBED_DIM)
    idx_t = sparse_input.T
    out6 = _gather_kernel(idx_t, table_rm)
    return out6.transpose(2, 4, 0, 1, 3).reshape(BATCH, NUM_FIELDS,
                                                 EMBED_DIM)


# speedup vs baseline: 1.4958x; 1.4952x over previous
import functools

import jax
import jax.numpy as jnp
from jax import lax
from jax.experimental import pallas as pl
from jax.experimental.pallas import tpu as pltpu
from jax.experimental.pallas import tpu_sc as plsc

BATCH = 4096
NUM_FIELDS = 26
EMBED_DIM = 16
VOCAB = 1000000

NUM_CORES = 2
NUM_SUBCORES = 16
NW = NUM_CORES * NUM_SUBCORES
CHUNK = 128
NBLK = BATCH // CHUNK

RUN = 1024
NFULL = VOCAB // RUN
TAIL_ROWS = 512
LAST64 = VOCAB - NFULL * RUN - TAIL_ROWS
RPW = NFULL // NW
NFREM = NFULL % NW

_mesh = plsc.VectorSubcoreMesh(core_axis_name="c", subcore_axis_name="s")


@functools.partial(
    pl.kernel,
    mesh=_mesh,
    out_type=jax.ShapeDtypeStruct((VOCAB * EMBED_DIM,), jnp.float32),
    scratch_types=[
        pltpu.VMEM((EMBED_DIM, RUN + 8), jnp.float32),
        pltpu.VMEM((EMBED_DIM, RUN + 8), jnp.float32),
        pltpu.VMEM((RUN * EMBED_DIM,), jnp.float32),
        pltpu.VMEM((RUN * EMBED_DIM,), jnp.float32),
        pltpu.SemaphoreType.DMA,
        pltpu.SemaphoreType.DMA,
        pltpu.SemaphoreType.DMA,
    ],
    compiler_params=pltpu.CompilerParams(use_tc_tiling_on_sc=True,
                                         needs_layout_passes=False),
)
def _transpose_kernel(tab_hbm, tail_hbm, out_hbm, in_a, in_b, ob_a, ob_b,
                      sem_a, sem_b, sem_o):
    w = lax.axis_index("s") * NUM_CORES + lax.axis_index("c")
    nruns = jnp.where(w < NFREM, RPW + 1, RPW)

    iota = lax.iota(jnp.int32, 16)

    def fire_load(k, ibuf, sem):
        r0 = (w + k * NW) * RUN
        return [
            pltpu.async_copy(tab_hbm.at[pl.ds(0, 8), pl.ds(r0, RUN)],
                             ibuf.at[pl.ds(0, 8), pl.ds(0, RUN)], sem),
            pltpu.async_copy(tab_hbm.at[pl.ds(8, 8), pl.ds(r0, RUN)],
                             ibuf.at[pl.ds(8, 8), pl.ds(0, RUN)], sem),
        ]

    i8 = iota & 7
    hi = iota >> 3
    vst_base = hi * EMBED_DIM + i8

    def transpose_run(ibuf, obuf, nrows):
        ib0 = ibuf.at[pl.ds(0, 8)]
        ib1 = ibuf.at[pl.ds(8, 8)]
        h = nrows // 2

        def body(j, carry):
            cvec, svec = carry
            ga0 = plsc.load_gather(ib0, [i8, cvec])
            ga1 = plsc.load_gather(ib1, [i8, cvec])
            gb0 = plsc.load_gather(ib0, [i8, cvec + h])
            gb1 = plsc.load_gather(ib1, [i8, cvec + h])
            plsc.store_scatter(obuf, [svec], ga0)
            plsc.store_scatter(obuf, [svec + 8], ga1)
            plsc.store_scatter(obuf, [svec + h * EMBED_DIM], gb0)
            plsc.store_scatter(obuf, [svec + h * EMBED_DIM + 8], gb1)
            return cvec + 2, svec + 2 * EMBED_DIM

        lax.fori_loop(0, nrows // 4, body, (hi, vst_base), unroll=4)

    def write_run(k, obuf, nrows):
        r0 = (w + k * NW) * RUN
        cp = pltpu.async_copy(
            obuf.at[pl.ds(0, nrows * EMBED_DIM)],
            out_hbm.at[pl.ds(r0 * EMBED_DIM, nrows * EMBED_DIM)],
            sem_o)
        cp.wait()

    def half(k, ibuf, obuf, sem_cur, ibuf_nxt, sem_nxt):
        @pl.when(k + 1 < nruns)
        def _():
            fire_load(k + 1, ibuf_nxt, sem_nxt)

        @pl.when(k < nruns)
        def _():
            pltpu.make_async_copy(
                tab_hbm.at[pl.ds(0, 8), pl.ds(0, RUN)],
                ibuf.at[pl.ds(0, 8), pl.ds(0, RUN)], sem_cur).wait()
            pltpu.make_async_copy(
                tab_hbm.at[pl.ds(8, 8), pl.ds(0, RUN)],
                ibuf.at[pl.ds(8, 8), pl.ds(0, RUN)], sem_cur).wait()
            transpose_run(ibuf, obuf, RUN)
            write_run(k, obuf, RUN)

    fire_load(0, in_a, sem_a)

    def loop(i, _):
        k = i * 2
        half(k, in_a, ob_a, sem_a, in_b, sem_b)
        half(k + 1, in_b, ob_b, sem_b, in_a, sem_a)
        return 0

    lax.fori_loop(0, (RPW + 2) // 2, loop, 0)

    @pl.when(w == NW - 1)
    def _tail():
        r0 = NFULL * RUN
        cps = [
            pltpu.async_copy(
                tab_hbm.at[pl.ds(0, 8), pl.ds(r0, TAIL_ROWS)],
                in_a.at[pl.ds(0, 8), pl.ds(0, TAIL_ROWS)], sem_a),
            pltpu.async_copy(
                tab_hbm.at[pl.ds(8, 8), pl.ds(r0, TAIL_ROWS)],
                in_a.at[pl.ds(8, 8), pl.ds(0, TAIL_ROWS)], sem_a),
        ]
        for cp in cps:
            cp.wait()
        transpose_run(in_a, ob_a, TAIL_ROWS)
        cp = pltpu.async_copy(
            ob_a.at[pl.ds(0, TAIL_ROWS * EMBED_DIM)],
            out_hbm.at[pl.ds(r0 * EMBED_DIM, TAIL_ROWS * EMBED_DIM)],
            sem_o)
        cp.wait()
        pltpu.sync_copy(tail_hbm, ob_b.at[pl.ds(0, LAST64 * EMBED_DIM)])
        pltpu.sync_copy(
            ob_b.at[pl.ds(0, LAST64 * EMBED_DIM)],
            out_hbm.at[pl.ds((VOCAB - LAST64) * EMBED_DIM,
                             LAST64 * EMBED_DIM)])


@functools.partial(
    pl.kernel,
    mesh=_mesh,
    out_type=jax.ShapeDtypeStruct((NUM_FIELDS, 2, NBLK, 8, CHUNK),
                                  jnp.float32),
    scratch_types=[
        pltpu.VMEM((NUM_FIELDS, CHUNK), jnp.int32),
        pltpu.VMEM((NUM_FIELDS, CHUNK, EMBED_DIM), jnp.float32),
        pltpu.VMEM((EMBED_DIM, CHUNK), jnp.float32),
        pltpu.SemaphoreType.DMA,
        pltpu.SemaphoreType.DMA,
    ],
    compiler_params=pltpu.CompilerParams(use_tc_tiling_on_sc=False,
                                         needs_layout_passes=False),
)
def _gather_kernel(idx_hbm, table_hbm, out_hbm,
                   idx_all, rows_all, obuf, sem_i, sem_g):
    w = lax.axis_index("s") * NUM_CORES + lax.axis_index("c")
    col = w * CHUNK

    idx_copies = [
        pltpu.async_copy(idx_hbm.at[f, pl.ds(col, CHUNK)],
                         idx_all.at[f], sem_i)
        for f in range(NUM_FIELDS)
    ]
    for cp in idx_copies:
        cp.wait()

    row_copies = [
        pltpu.async_copy(table_hbm.at[idx_all.at[f]],
                         rows_all.at[f], sem_g)
        for f in range(NUM_FIELDS)
    ]
    for cp in row_copies:
        cp.wait()

    lanes = lax.iota(jnp.int32, 16)

    def field_body(f, _):
        def col_body(c, _):
            cvec = jnp.zeros((16,), jnp.int32) + c
            for k in range(CHUNK // 16):
                v = plsc.load_gather(rows_all.at[f],
                                     [k * 16 + lanes, cvec])
                obuf[c, pl.ds(k * 16, 16)] = v
            return 0
        lax.fori_loop(0, EMBED_DIM, col_body, 0)
        pltpu.sync_copy(obuf.at[pl.ds(0, 8)], out_hbm.at[f, 0, w])
        pltpu.sync_copy(obuf.at[pl.ds(8, 8)], out_hbm.at[f, 1, w])
        return 0

    lax.fori_loop(0, NUM_FIELDS, field_body, 0)


def kernel(sparse_input, table):
    tail_flat = table[VOCAB - LAST64:].reshape(LAST64 * EMBED_DIM)
    table_rm = _transpose_kernel(table.T, tail_flat).reshape(VOCAB,
                                                             EMBED_DIM)
    idx_t = sparse_input.T
    out6 = _gather_kernel(idx_t, table_rm)
    return out6.transpose(2, 4, 0, 1, 3).reshape(BATCH, NUM_FIELDS,
                                                 EMBED_DIM)


# speedup vs baseline: 1.5496x; 1.0360x over previous
import functools

import jax
import jax.numpy as jnp
from jax import lax
from jax.experimental import pallas as pl
from jax.experimental.pallas import tpu as pltpu
from jax.experimental.pallas import tpu_sc as plsc

BATCH = 4096
NUM_FIELDS = 26
EMBED_DIM = 16
VOCAB = 1000000

NUM_CORES = 2
NUM_SUBCORES = 16
NW = NUM_CORES * NUM_SUBCORES
CHUNK = 128
NBLK = BATCH // CHUNK

RUN = 1024
NFULL = VOCAB // RUN
TAIL_ROWS = 512
LAST64 = VOCAB - NFULL * RUN - TAIL_ROWS
RPW = NFULL // NW
NFREM = NFULL % NW

_mesh = plsc.VectorSubcoreMesh(core_axis_name="c", subcore_axis_name="s")


@functools.partial(
    pl.kernel,
    mesh=_mesh,
    out_type=jax.ShapeDtypeStruct((VOCAB * EMBED_DIM,), jnp.float32),
    scratch_types=[
        pltpu.VMEM((EMBED_DIM, RUN + 8), jnp.float32),
        pltpu.VMEM((EMBED_DIM, RUN + 8), jnp.float32),
        pltpu.VMEM((RUN * EMBED_DIM,), jnp.float32),
        pltpu.VMEM((RUN * EMBED_DIM,), jnp.float32),
        pltpu.SemaphoreType.DMA,
        pltpu.SemaphoreType.DMA,
        pltpu.SemaphoreType.DMA,
    ],
    compiler_params=pltpu.CompilerParams(use_tc_tiling_on_sc=True,
                                         needs_layout_passes=False),
)
def _transpose_kernel(tab_hbm, tail_hbm, out_hbm, in_a, in_b, ob_a, ob_b,
                      sem_a, sem_b, sem_o):
    w = lax.axis_index("s") * NUM_CORES + lax.axis_index("c")
    nruns = jnp.where(w < NFREM, RPW + 1, RPW)

    iota = lax.iota(jnp.int32, 16)

    def fire_load(k, ibuf, sem):
        r0 = (w + k * NW) * RUN
        return [
            pltpu.async_copy(tab_hbm.at[pl.ds(0, 8), pl.ds(r0, RUN)],
                             ibuf.at[pl.ds(0, 8), pl.ds(0, RUN)], sem),
            pltpu.async_copy(tab_hbm.at[pl.ds(8, 8), pl.ds(r0, RUN)],
                             ibuf.at[pl.ds(8, 8), pl.ds(0, RUN)], sem),
        ]

    i8 = iota & 7
    hi = iota >> 3
    vst_base = hi * EMBED_DIM + i8

    def transpose_run(ibuf, obuf, nrows):
        ib0 = ibuf.at[pl.ds(0, 8)]
        ib1 = ibuf.at[pl.ds(8, 8)]
        h = nrows // 4

        def body(j, carry):
            cvec, svec = carry
            gs = []
            for q in range(4):
                g0 = plsc.load_gather(ib0, [i8, cvec + q * h])
                g1 = plsc.load_gather(ib1, [i8, cvec + q * h])
                gs.append((g0, g1))
            for q, (g0, g1) in enumerate(gs):
                plsc.store_scatter(obuf, [svec + q * h * EMBED_DIM], g0)
                plsc.store_scatter(obuf,
                                   [svec + q * h * EMBED_DIM + 8], g1)
            return cvec + 2, svec + 2 * EMBED_DIM

        lax.fori_loop(0, nrows // 8, body, (hi, vst_base), unroll=2)

    def write_run(k, obuf, nrows):
        r0 = (w + k * NW) * RUN
        cp = pltpu.async_copy(
            obuf.at[pl.ds(0, nrows * EMBED_DIM)],
            out_hbm.at[pl.ds(r0 * EMBED_DIM, nrows * EMBED_DIM)],
            sem_o)
        cp.wait()

    def half(k, ibuf, obuf, sem_cur, ibuf_nxt, sem_nxt):
        @pl.when(k + 1 < nruns)
        def _():
            fire_load(k + 1, ibuf_nxt, sem_nxt)

        @pl.when(k < nruns)
        def _():
            pltpu.make_async_copy(
                tab_hbm.at[pl.ds(0, 8), pl.ds(0, RUN)],
                ibuf.at[pl.ds(0, 8), pl.ds(0, RUN)], sem_cur).wait()
            pltpu.make_async_copy(
                tab_hbm.at[pl.ds(8, 8), pl.ds(0, RUN)],
                ibuf.at[pl.ds(8, 8), pl.ds(0, RUN)], sem_cur).wait()
            transpose_run(ibuf, obuf, RUN)
            write_run(k, obuf, RUN)

    fire_load(0, in_a, sem_a)

    def loop(i, _):
        k = i * 2
        half(k, in_a, ob_a, sem_a, in_b, sem_b)
        half(k + 1, in_b, ob_b, sem_b, in_a, sem_a)
        return 0

    lax.fori_loop(0, (RPW + 2) // 2, loop, 0)

    @pl.when(w == NW - 1)
    def _tail():
        r0 = NFULL * RUN
        cps = [
            pltpu.async_copy(
                tab_hbm.at[pl.ds(0, 8), pl.ds(r0, TAIL_ROWS)],
                in_a.at[pl.ds(0, 8), pl.ds(0, TAIL_ROWS)], sem_a),
            pltpu.async_copy(
                tab_hbm.at[pl.ds(8, 8), pl.ds(r0, TAIL_ROWS)],
                in_a.at[pl.ds(8, 8), pl.ds(0, TAIL_ROWS)], sem_a),
        ]
        for cp in cps:
            cp.wait()
        transpose_run(in_a, ob_a, TAIL_ROWS)
        cp = pltpu.async_copy(
            ob_a.at[pl.ds(0, TAIL_ROWS * EMBED_DIM)],
            out_hbm.at[pl.ds(r0 * EMBED_DIM, TAIL_ROWS * EMBED_DIM)],
            sem_o)
        cp.wait()
        pltpu.sync_copy(tail_hbm, ob_b.at[pl.ds(0, LAST64 * EMBED_DIM)])
        pltpu.sync_copy(
            ob_b.at[pl.ds(0, LAST64 * EMBED_DIM)],
            out_hbm.at[pl.ds((VOCAB - LAST64) * EMBED_DIM,
                             LAST64 * EMBED_DIM)])


@functools.partial(
    pl.kernel,
    mesh=_mesh,
    out_type=jax.ShapeDtypeStruct((NUM_FIELDS, 2, NBLK, 8, CHUNK),
                                  jnp.float32),
    scratch_types=[
        pltpu.VMEM((NUM_FIELDS, CHUNK), jnp.int32),
        pltpu.VMEM((NUM_FIELDS, CHUNK, EMBED_DIM), jnp.float32),
        pltpu.VMEM((EMBED_DIM, CHUNK), jnp.float32),
        pltpu.SemaphoreType.DMA,
        pltpu.SemaphoreType.DMA,
    ],
    compiler_params=pltpu.CompilerParams(use_tc_tiling_on_sc=False,
                                         needs_layout_passes=False),
)
def _gather_kernel(idx_hbm, table_hbm, out_hbm,
                   idx_all, rows_all, obuf, sem_i, sem_g):
    w = lax.axis_index("s") * NUM_CORES + lax.axis_index("c")
    col = w * CHUNK

    idx_copies = [
        pltpu.async_copy(idx_hbm.at[f, pl.ds(col, CHUNK)],
                         idx_all.at[f], sem_i)
        for f in range(NUM_FIELDS)
    ]
    for cp in idx_copies:
        cp.wait()

    row_copies = [
        pltpu.async_copy(table_hbm.at[idx_all.at[f]],
                         rows_all.at[f], sem_g)
        for f in range(NUM_FIELDS)
    ]
    for cp in row_copies:
        cp.wait()

    lanes = lax.iota(jnp.int32, 16)

    def field_body(f, _):
        def col_body(c, _):
            cvec = jnp.zeros((16,), jnp.int32) + c
            for k in range(CHUNK // 16):
                v = plsc.load_gather(rows_all.at[f],
                                     [k * 16 + lanes, cvec])
                obuf[c, pl.ds(k * 16, 16)] = v
            return 0
        lax.fori_loop(0, EMBED_DIM, col_body, 0)
        pltpu.sync_copy(obuf.at[pl.ds(0, 8)], out_hbm.at[f, 0, w])
        pltpu.sync_copy(obuf.at[pl.ds(8, 8)], out_hbm.at[f, 1, w])
        return 0

    lax.fori_loop(0, NUM_FIELDS, field_body, 0)


def kernel(sparse_input, table):
    tail_flat = table[VOCAB - LAST64:].reshape(LAST64 * EMBED_DIM)
    table_rm = _transpose_kernel(table.T, tail_flat).reshape(VOCAB,
                                                             EMBED_DIM)
    idx_t = sparse_input.T
    out6 = _gather_kernel(idx_t, table_rm)
    return out6.transpose(2, 4, 0, 1, 3).reshape(BATCH, NUM_FIELDS,
                                                 EMBED_DIM)


# speedup vs baseline: 1.9450x; 1.2552x over previous
import functools

import jax
import jax.numpy as jnp
from jax import lax
from jax.experimental import pallas as pl
from jax.experimental.pallas import tpu as pltpu
from jax.experimental.pallas import tpu_sc as plsc

BATCH = 4096
NUM_FIELDS = 26
EMBED_DIM = 16
VOCAB = 1000000

NUM_CORES = 2
NUM_SUBCORES = 16
NW = NUM_CORES * NUM_SUBCORES
CHUNK = 128
NBLK = BATCH // CHUNK

RUN = 1024
NFULL = VOCAB // RUN
TAIL_ROWS = 512
LAST64 = VOCAB - NFULL * RUN - TAIL_ROWS
RPW = NFULL // NW
NFREM = NFULL % NW

_mesh = plsc.VectorSubcoreMesh(core_axis_name="c", subcore_axis_name="s")


@functools.partial(
    pl.kernel,
    mesh=_mesh,
    out_type=jax.ShapeDtypeStruct((VOCAB * EMBED_DIM,), jnp.float32),
    scratch_types=[
        pltpu.VMEM((EMBED_DIM, RUN + 8), jnp.float32),
        pltpu.VMEM((EMBED_DIM, RUN + 8), jnp.float32),
        pltpu.VMEM((RUN * EMBED_DIM,), jnp.float32),
        pltpu.VMEM((RUN * EMBED_DIM,), jnp.float32),
        pltpu.SemaphoreType.DMA,
        pltpu.SemaphoreType.DMA,
        pltpu.SemaphoreType.DMA,
    ],
    compiler_params=pltpu.CompilerParams(use_tc_tiling_on_sc=True,
                                         needs_layout_passes=False),
)
def _transpose_kernel(tab_hbm, tail_hbm, out_hbm, in_a, in_b, ob_a, ob_b,
                      sem_a, sem_b, sem_o):
    w = lax.axis_index("s") * NUM_CORES + lax.axis_index("c")
    nruns = jnp.where(w < NFREM, RPW + 1, RPW)

    iota = lax.iota(jnp.int32, 16)

    def fire_load(k, ibuf, sem):
        r0 = (w + k * NW) * RUN
        return [
            pltpu.async_copy(tab_hbm.at[pl.ds(0, 8), pl.ds(r0, RUN)],
                             ibuf.at[pl.ds(0, 8), pl.ds(0, RUN)], sem),
            pltpu.async_copy(tab_hbm.at[pl.ds(8, 8), pl.ds(r0, RUN)],
                             ibuf.at[pl.ds(8, 8), pl.ds(0, RUN)], sem),
        ]

    i8 = iota & 7
    hi = iota >> 3
    vst_base = hi * EMBED_DIM + i8

    def transpose_run(ibuf, obuf, nrows):
        iota16x = iota * EMBED_DIM

        def body(i, _):
            base = i * (16 * EMBED_DIM)
            for cl in range(EMBED_DIM):
                v = ibuf[cl, pl.ds(i * 16, 16)]
                plsc.store_scatter(obuf, [iota16x + (base + cl)], v)
            return 0

        lax.fori_loop(0, nrows // 16, body, 0, unroll=2)

    def write_run(k, obuf, nrows):
        r0 = (w + k * NW) * RUN
        cp = pltpu.async_copy(
            obuf.at[pl.ds(0, nrows * EMBED_DIM)],
            out_hbm.at[pl.ds(r0 * EMBED_DIM, nrows * EMBED_DIM)],
            sem_o)
        cp.wait()

    def half(k, ibuf, obuf, sem_cur, ibuf_nxt, sem_nxt):
        @pl.when(k + 1 < nruns)
        def _():
            fire_load(k + 1, ibuf_nxt, sem_nxt)

        @pl.when(k < nruns)
        def _():
            pltpu.make_async_copy(
                tab_hbm.at[pl.ds(0, 8), pl.ds(0, RUN)],
                ibuf.at[pl.ds(0, 8), pl.ds(0, RUN)], sem_cur).wait()
            pltpu.make_async_copy(
                tab_hbm.at[pl.ds(8, 8), pl.ds(0, RUN)],
                ibuf.at[pl.ds(8, 8), pl.ds(0, RUN)], sem_cur).wait()
            transpose_run(ibuf, obuf, RUN)
            write_run(k, obuf, RUN)

    fire_load(0, in_a, sem_a)

    def loop(i, _):
        k = i * 2
        half(k, in_a, ob_a, sem_a, in_b, sem_b)
        half(k + 1, in_b, ob_b, sem_b, in_a, sem_a)
        return 0

    lax.fori_loop(0, (RPW + 2) // 2, loop, 0)

    @pl.when(w == NW - 1)
    def _tail():
        r0 = NFULL * RUN
        cps = [
            pltpu.async_copy(
                tab_hbm.at[pl.ds(0, 8), pl.ds(r0, TAIL_ROWS)],
                in_a.at[pl.ds(0, 8), pl.ds(0, TAIL_ROWS)], sem_a),
            pltpu.async_copy(
                tab_hbm.at[pl.ds(8, 8), pl.ds(r0, TAIL_ROWS)],
                in_a.at[pl.ds(8, 8), pl.ds(0, TAIL_ROWS)], sem_a),
        ]
        for cp in cps:
            cp.wait()
        transpose_run(in_a, ob_a, TAIL_ROWS)
        cp = pltpu.async_copy(
            ob_a.at[pl.ds(0, TAIL_ROWS * EMBED_DIM)],
            out_hbm.at[pl.ds(r0 * EMBED_DIM, TAIL_ROWS * EMBED_DIM)],
            sem_o)
        cp.wait()
        pltpu.sync_copy(tail_hbm, ob_b.at[pl.ds(0, LAST64 * EMBED_DIM)])
        pltpu.sync_copy(
            ob_b.at[pl.ds(0, LAST64 * EMBED_DIM)],
            out_hbm.at[pl.ds((VOCAB - LAST64) * EMBED_DIM,
                             LAST64 * EMBED_DIM)])


@functools.partial(
    pl.kernel,
    mesh=_mesh,
    out_type=jax.ShapeDtypeStruct((NUM_FIELDS, 2, NBLK, 8, CHUNK),
                                  jnp.float32),
    scratch_types=[
        pltpu.VMEM((NUM_FIELDS, CHUNK), jnp.int32),
        pltpu.VMEM((NUM_FIELDS, CHUNK, EMBED_DIM), jnp.float32),
        pltpu.VMEM((EMBED_DIM, CHUNK), jnp.float32),
        pltpu.SemaphoreType.DMA,
        pltpu.SemaphoreType.DMA,
    ],
    compiler_params=pltpu.CompilerParams(use_tc_tiling_on_sc=False,
                                         needs_layout_passes=False),
)
def _gather_kernel(idx_hbm, table_hbm, out_hbm,
                   idx_all, rows_all, obuf, sem_i, sem_g):
    w = lax.axis_index("s") * NUM_CORES + lax.axis_index("c")
    col = w * CHUNK

    idx_copies = [
        pltpu.async_copy(idx_hbm.at[f, pl.ds(col, CHUNK)],
                         idx_all.at[f], sem_i)
        for f in range(NUM_FIELDS)
    ]
    for cp in idx_copies:
        cp.wait()

    row_copies = [
        pltpu.async_copy(table_hbm.at[idx_all.at[f]],
                         rows_all.at[f], sem_g)
        for f in range(NUM_FIELDS)
    ]
    for cp in row_copies:
        cp.wait()

    lanes = lax.iota(jnp.int32, 16)

    def field_body(f, _):
        def col_body(c, _):
            cvec = jnp.zeros((16,), jnp.int32) + c
            for k in range(CHUNK // 16):
                v = plsc.load_gather(rows_all.at[f],
                                     [k * 16 + lanes, cvec])
                obuf[c, pl.ds(k * 16, 16)] = v
            return 0
        lax.fori_loop(0, EMBED_DIM, col_body, 0)
        pltpu.sync_copy(obuf.at[pl.ds(0, 8)], out_hbm.at[f, 0, w])
        pltpu.sync_copy(obuf.at[pl.ds(8, 8)], out_hbm.at[f, 1, w])
        return 0

    lax.fori_loop(0, NUM_FIELDS, field_body, 0)


def kernel(sparse_input, table):
    tail_flat = table[VOCAB - LAST64:].reshape(LAST64 * EMBED_DIM)
    table_rm = _transpose_kernel(table.T, tail_flat).reshape(VOCAB,
                                                             EMBED_DIM)
    idx_t = sparse_input.T
    out6 = _gather_kernel(idx_t, table_rm)
    return out6.transpose(2, 4, 0, 1, 3).reshape(BATCH, NUM_FIELDS,
                                                 EMBED_DIM)
